# baseline (speedup 1.0000x reference)
import functools

import jax
import jax.numpy as jnp
from jax import lax
from jax.experimental import pallas as pl
from jax.experimental.pallas import tpu as pltpu
from jax.experimental.pallas import tpu_sc as plsc

_N = 10000
_E = 320000
_H = 1024
_CH = 8
_NPAD = 10240
_RPS = _NPAD // 16
_BATCH = 128
_NB0 = 80
_NBD = 160
_GRP = 8
_R = 1
_NSLOT = 2
_BN = 400
_EPS = 1e-5


def _sc_mesh():
    return plsc.VectorSubcoreMesh(core_axis_name="c", subcore_axis_name="s")


def _pipelined_group(x_hbm, src_v, dst_v, rows_v, acc_sh, gsems, ssems):
    nsub = _GRP
    gh = [None] * nsub
    sh = [None] * nsub

    def fire_g(t):
        return pltpu.async_copy(x_hbm.at[src_v.at[t]],
                                rows_v.at[t % _NSLOT], gsems[t % _NSLOT])

    def fire_s(t):
        return pltpu.async_copy(rows_v.at[t % _NSLOT],
                                acc_sh.at[dst_v.at[t]], ssems[t % _NSLOT],
                                add=True)

    for t in range(min(_NSLOT, nsub)):
        gh[t] = fire_g(t)
    for t in range(nsub):
        gh[t].wait()
        sh[t] = fire_s(t)
        v = t - (_NSLOT - 1)
        if v >= 0:
            sh[v].wait()
            if v + _NSLOT < nsub:
                gh[v + _NSLOT] = fire_g(v + _NSLOT)
    for t in range(max(0, nsub - (_NSLOT - 1)), nsub):
        sh[t].wait()


def _sc_layer0(x, src_a, dst_a, zrows, ones128):
    @functools.partial(
        pl.kernel,
        out_type=(
            jax.ShapeDtypeStruct((2, _N, 128), jnp.float32),
            jax.ShapeDtypeStruct((2, _N, 128), jnp.float32),
        ),
        mesh=_sc_mesh(),
        scratch_types=[
            pltpu.VMEM((_GRP, _BATCH), jnp.int32),
            pltpu.VMEM((_GRP, _BATCH), jnp.int32),
            pltpu.VMEM((_NSLOT * _R, _BATCH, 128), jnp.float32),
            pltpu.VMEM_SHARED((_NPAD, 128), jnp.float32),
            pltpu.SemaphoreType.DMA,
            pltpu.SemaphoreType.DMA,
            pltpu.SemaphoreType.DMA,
            pltpu.SemaphoreType.DMA,
        ],
    )
    def k(x_hbm, src_hbm, dst_hbm, zr_hbm, o_hbm,
          agg_out, cnt_out, src_v, dst_v, rows_v,
          acc_sh, gsA, gsB, ssA, ssB):
        c = lax.axis_index("c")
        s = lax.axis_index("s")
        wid = c * 16 + s
        r0 = s * _RPS

        pltpu.sync_copy(zr_hbm, acc_sh.at[pl.ds(r0, _RPS)])
        plsc.subcore_barrier()

        def group(g, carry):
            pltpu.sync_copy(src_hbm.at[wid, pl.ds(g * _GRP, _GRP)], src_v)
            pltpu.sync_copy(dst_hbm.at[wid, pl.ds(g * _GRP, _GRP)], dst_v)
            _pipelined_group(x_hbm, src_v, dst_v, rows_v, acc_sh,
                             (gsA, gsB), (ssA, ssB))
            return carry

        lax.fori_loop(0, _NB0 // _GRP, group, 0)
        plsc.subcore_barrier()

        @pl.when(s < 15)
        def _():
            pltpu.sync_copy(acc_sh.at[pl.ds(r0, _RPS)],
                            agg_out.at[c, pl.ds(r0, _RPS)])

        @pl.when(s == 15)
        def _():
            pltpu.sync_copy(acc_sh.at[pl.ds(15 * _RPS, _N - 15 * _RPS)],
                            agg_out.at[c, pl.ds(15 * _RPS, _N - 15 * _RPS)])

        pltpu.sync_copy(zr_hbm, acc_sh.at[pl.ds(r0, _RPS)])
        pltpu.sync_copy(o_hbm, rows_v.at[0])
        plsc.subcore_barrier()

        def group_cnt(g, carry):
            pltpu.sync_copy(dst_hbm.at[wid, pl.ds(g * _GRP, _GRP)], dst_v)
            for b in range(_GRP):
                pltpu.sync_copy(rows_v.at[0], acc_sh.at[dst_v.at[b]],
                                add=True)
            return carry

        lax.fori_loop(0, _NB0 // _GRP, group_cnt, 0)
        plsc.subcore_barrier()

        @pl.when(s < 15)
        def _():
            pltpu.sync_copy(acc_sh.at[pl.ds(r0, _RPS)],
                            cnt_out.at[c, pl.ds(r0, _RPS)])

        @pl.when(s == 15)
        def _():
            pltpu.sync_copy(acc_sh.at[pl.ds(15 * _RPS, _N - 15 * _RPS)],
                            cnt_out.at[c, pl.ds(15 * _RPS, _N - 15 * _RPS)])

    return k(x, src_a, dst_a, zrows, ones128)


def _sc_spmm(h2d, src_adj, dst_d, zrows):
    @functools.partial(
        pl.kernel,
        out_type=jax.ShapeDtypeStruct((_CH, _N, 128), jnp.float32),
        mesh=_sc_mesh(),
        scratch_types=[
            pltpu.VMEM((_GRP, _BATCH), jnp.int32),
            pltpu.VMEM((_GRP, _BATCH), jnp.int32),
            pltpu.VMEM((_NSLOT * _R, _BATCH, 128), jnp.float32),
            pltpu.VMEM_SHARED((_NPAD, 128), jnp.float32),
            pltpu.SemaphoreType.DMA,
            pltpu.SemaphoreType.DMA,
            pltpu.SemaphoreType.DMA,
            pltpu.SemaphoreType.DMA,
        ],
    )
    def k(h_hbm, src_hbm, dst_hbm, zr_hbm, agg_out,
          src_v, dst_v, rows_v, acc_sh, gsA, gsB, ssA, ssB):
        c = lax.axis_index("c")
        s = lax.axis_index("s")
        r0 = s * _RPS

        for kk in range(_CH // 2):
            ci = c * (_CH // 2) + kk
            pltpu.sync_copy(zr_hbm, acc_sh.at[pl.ds(r0, _RPS)])
            plsc.subcore_barrier()

            def group(g, carry):
                pltpu.sync_copy(src_hbm.at[ci, s, pl.ds(g * _GRP, _GRP)],
                                src_v)
                pltpu.sync_copy(dst_hbm.at[s, pl.ds(g * _GRP, _GRP)], dst_v)
                _pipelined_group(h_hbm, src_v, dst_v, rows_v, acc_sh,
                                 (gsA, gsB), (ssA, ssB))
                return carry

            lax.fori_loop(0, _NBD // _GRP, group, 0)
            plsc.subcore_barrier()

            @pl.when(s < 15)
            def _():
                pltpu.sync_copy(acc_sh.at[pl.ds(r0, _RPS)],
                                agg_out.at[ci, pl.ds(r0, _RPS)])

            @pl.when(s == 15)
            def _():
                pltpu.sync_copy(acc_sh.at[pl.ds(15 * _RPS, _N - 15 * _RPS)],
                                agg_out.at[ci, pl.ds(15 * _RPS, _N - 15 * _RPS)])

    return k(h2d, src_adj, dst_d, zrows)


def _ln_leaky(h, g, b):
    mu = jnp.mean(h, axis=-1, keepdims=True)
    var = jnp.mean((h - mu) ** 2, axis=-1, keepdims=True)
    h = (h - mu) * lax.rsqrt(var + _EPS) * g + b
    return jnp.where(h >= 0, h, 0.01 * h)


def _tc_layer0(aggp, cntp, w, b, vn, g, be):
    def body(p_ref, c_ref, w_ref, b_ref, vn_ref, g_ref, be_ref,
             h_ref, cs_ref, rd_ref):
        i = pl.program_id(0)
        agg = p_ref[0] + p_ref[1]
        cnt = c_ref[0, :, 0:1] + c_ref[1, :, 0:1]
        rdeg = 1.0 / jnp.maximum(cnt, 1.0)
        h = jnp.dot(agg * rdeg, w_ref[...], preferred_element_type=jnp.float32)
        vnm = jnp.mean(vn_ref[...], axis=0, keepdims=True)
        h = _ln_leaky(h + b_ref[...] + vnm, g_ref[...], be_ref[...])
        for cc in range(_CH):
            h_ref[cc] = h[:, cc * 128:(cc + 1) * 128]
        rd_ref[...] = jnp.broadcast_to(rdeg, (_BN, 128))

        @pl.when(i == 0)
        def _():
            cs_ref[...] = jnp.zeros((1, _H), jnp.float32)

        cs_ref[...] += jnp.sum(h, axis=0, keepdims=True)

    return pl.pallas_call(
        body,
        grid=(_N // _BN,),
        in_specs=[
            pl.BlockSpec((2, _BN, 128), lambda i: (0, i, 0)),
            pl.BlockSpec((2, _BN, 128), lambda i: (0, i, 0)),
            pl.BlockSpec((128, _H), lambda i: (0, 0)),
            pl.BlockSpec((1, _H), lambda i: (0, 0)),
            pl.BlockSpec((3, _H), lambda i: (0, 0)),
            pl.BlockSpec((1, _H), lambda i: (0, 0)),
            pl.BlockSpec((1, _H), lambda i: (0, 0)),
        ],
        out_specs=[
            pl.BlockSpec((_CH, _BN, 128), lambda i: (0, i, 0)),
            pl.BlockSpec((1, _H), lambda i: (0, 0)),
            pl.BlockSpec((_BN, 128), lambda i: (i, 0)),
        ],
        out_shape=[
            jax.ShapeDtypeStruct((_CH, _N, 128), jnp.float32),
            jax.ShapeDtypeStruct((1, _H), jnp.float32),
            jax.ShapeDtypeStruct((_N, 128), jnp.float32),
        ],
    )(aggp, cntp, w, b, vn, g, be)


def _tc_hidden(agg, rdeg, w, b, vn, g, be, final, out_w=None, out_b=None):
    def body(*refs):
        if final:
            (a_ref, rd_ref, w_ref, b_ref, vn_ref, g_ref, be_ref,
             ow_ref, ob_ref, y_ref) = refs
        else:
            (a_ref, rd_ref, w_ref, b_ref, vn_ref, g_ref, be_ref,
             h_ref, cs_ref) = refs
        i = pl.program_id(0)
        rd = rd_ref[:, 0:1]
        h = jnp.zeros((_BN, _H), jnp.float32)
        for cc in range(_CH):
            h += jnp.dot(a_ref[cc] * rd, w_ref[cc * 128:(cc + 1) * 128, :],
                         preferred_element_type=jnp.float32)
        vnm = jnp.mean(vn_ref[...], axis=0, keepdims=True)
        h = _ln_leaky(h + b_ref[...] + vnm, g_ref[...], be_ref[...])
        if final:
            y_ref[...] = (jnp.dot(h, ow_ref[...],
                                  preferred_element_type=jnp.float32)
                          + ob_ref[...])
        else:
            for cc in range(_CH):
                h_ref[cc] = h[:, cc * 128:(cc + 1) * 128]

            @pl.when(i == 0)
            def _():
                cs_ref[...] = jnp.zeros((1, _H), jnp.float32)

            cs_ref[...] += jnp.sum(h, axis=0, keepdims=True)

    in_specs = [
        pl.BlockSpec((_CH, _BN, 128), lambda i: (0, i, 0)),
        pl.BlockSpec((_BN, 128), lambda i: (i, 0)),
        pl.BlockSpec((_H, _H), lambda i: (0, 0)),
        pl.BlockSpec((1, _H), lambda i: (0, 0)),
        pl.BlockSpec((3, _H), lambda i: (0, 0)),
        pl.BlockSpec((1, _H), lambda i: (0, 0)),
        pl.BlockSpec((1, _H), lambda i: (0, 0)),
    ]
    args = [agg, rdeg, w, b, vn, g, be]
    if final:
        in_specs += [pl.BlockSpec((_H, 128), lambda i: (0, 0)),
                     pl.BlockSpec((1, 128), lambda i: (0, 0))]
        args += [out_w, out_b]
        out_specs = pl.BlockSpec((_BN, 128), lambda i: (i, 0))
        out_shape = jax.ShapeDtypeStruct((_N, 128), jnp.float32)
    else:
        out_specs = [
            pl.BlockSpec((_CH, _BN, 128), lambda i: (0, i, 0)),
            pl.BlockSpec((1, _H), lambda i: (0, 0)),
        ]
        out_shape = [
            jax.ShapeDtypeStruct((_CH, _N, 128), jnp.float32),
            jax.ShapeDtypeStruct((1, _H), jnp.float32),
        ]
    return pl.pallas_call(
        body, grid=(_N // _BN,), in_specs=in_specs,
        out_specs=out_specs, out_shape=out_shape,
    )(*args)


def _tc_vn_update(cs, vn, w1c, b1r, g1, be1, w2c, b2r, g2, be2):
    def body(cs_ref, vn_ref, w1_ref, b1_ref, g1_ref, be1_ref,
             w2_ref, b2_ref, g2_ref, be2_ref, o_ref):
        gi = cs_ref[...] * (1.0 / _N)
        gmat = vn_ref[...] + gi
        t_all = jnp.dot(gmat, w1_ref[...], preferred_element_type=jnp.float32)
        rows = lax.broadcasted_iota(jnp.int32, (3, 1), 0)
        t = jnp.zeros((3, 128), jnp.float32)
        for v in range(3):
            m = jnp.where(rows == v, 1.0, 0.0)
            t += t_all[:, v * 128:(v + 1) * 128] * m
        t = t + b1_ref[...]
        mu = jnp.mean(t, axis=-1, keepdims=True)
        var = jnp.mean((t - mu) ** 2, axis=-1, keepdims=True)
        t = (t - mu) * lax.rsqrt(var + _EPS) * g1_ref[...] + be1_ref[...]
        t = jnp.maximum(t, 0.0)
        z = jnp.concatenate(
            [t * jnp.where(rows == v, 1.0, 0.0) for v in range(3)], axis=1)
        u = jnp.dot(z, w2_ref[...], preferred_element_type=jnp.float32)
        u = u + b2_ref[...]
        mu = jnp.mean(u, axis=-1, keepdims=True)
        var = jnp.mean((u - mu) ** 2, axis=-1, keepdims=True)
        u = (u - mu) * lax.rsqrt(var + _EPS) * g2_ref[...] + be2_ref[...]
        u = jnp.maximum(u, 0.0)
        o_ref[...] = vn_ref[...] + u

    return pl.pallas_call(
        body,
        out_shape=jax.ShapeDtypeStruct((3, _H), jnp.float32),
    )(cs, vn, w1c, b1r, g1, be1, w2c, b2r, g2, be2)


def _pad_edges(src, dst, n_workers, nb):
    total = n_workers * nb * _BATCH
    pad = total - _E
    psrc = (jnp.arange(pad, dtype=jnp.int32) * 997) % _N
    pdst = _N + (jnp.arange(pad, dtype=jnp.int32) % 16)
    src_p = jnp.concatenate([src, psrc]).reshape(n_workers, nb, _BATCH)
    dst_p = jnp.concatenate([dst, pdst]).reshape(n_workers, nb, _BATCH)
    return src_p, dst_p


def kernel(x, edge_index, params):
    src = edge_index[0].astype(jnp.int32)
    dst = edge_index[1].astype(jnp.int32)
    order = jnp.argsort(src)
    src = src[order]
    dst = dst[order]

    src_a, dst_a = _pad_edges(src, dst, 32, _NB0)
    src_d, dst_d = _pad_edges(src, dst, 16, _NBD)
    chunk_off = (jnp.arange(_CH, dtype=jnp.int32) * _N).reshape(_CH, 1, 1, 1)
    src_adj = src_d[None] + chunk_off

    zrows = jnp.zeros((_RPS, 128), jnp.float32)
    ones128 = jnp.ones((_BATCH, 128), jnp.float32)

    conv = params["conv"]
    ln = params["ln"]
    vn = params["vn_emb"]
    row = lambda a: a.reshape(1, -1)

    mlp_stacks = []
    for i in range(3):
        ms = params["mlps"][i]
        w1c = jnp.concatenate([m["W1"] for m in ms], axis=1)
        b1r = jnp.stack([m["b1"] for m in ms], axis=0)
        g1 = jnp.stack([m["g1"] for m in ms], axis=0)
        be1 = jnp.stack([m["be1"] for m in ms], axis=0)
        w2c = jnp.concatenate([m["W2"] for m in ms], axis=0)
        b2r = jnp.stack([m["b2"] for m in ms], axis=0)
        g2 = jnp.stack([m["g2"] for m in ms], axis=0)
        be2 = jnp.stack([m["be2"] for m in ms], axis=0)
        mlp_stacks.append((w1c, b1r, g1, be1, w2c, b2r, g2, be2))

    aggp, cntp = _sc_layer0(x, src_a, dst_a, zrows, ones128)
    hc, cs, rdeg = _tc_layer0(aggp, cntp, conv[0]["W"], row(conv[0]["b"]),
                              vn, row(ln[0]["g"]), row(ln[0]["b"]))
    vn = _tc_vn_update(cs, vn, *mlp_stacks[0])

    for i in (1, 2):
        agg = _sc_spmm(hc.reshape(_CH * _N, 128), src_adj, dst_d, zrows)
        hc, cs = _tc_hidden(agg, rdeg, conv[i]["W"], row(conv[i]["b"]),
                            vn, row(ln[i]["g"]), row(ln[i]["b"]), final=False)
        vn = _tc_vn_update(cs, vn, *mlp_stacks[i])

    agg = _sc_spmm(hc.reshape(_CH * _N, 128), src_adj, dst_d, zrows)
    return _tc_hidden(agg, rdeg, conv[3]["W"], row(conv[3]["b"]),
                      vn, row(ln[3]["g"]), row(ln[3]["b"]), final=True,
                      out_w=params["out_W"], out_b=row(params["out_b"]))


# speedup vs baseline: 2.8151x; 2.8151x over previous
import functools

import jax
import jax.numpy as jnp
from jax import lax
from jax.experimental import pallas as pl
from jax.experimental.pallas import tpu as pltpu
from jax.experimental.pallas import tpu_sc as plsc

_N = 10000
_E = 320000
_H = 1024
_CH = 8
_NPAD = 10240
_RPS = _NPAD // 16
_BATCH = 128
_NB0 = 80
_NBD = 160
_GRP = 16
_R = 1
_NSLOT = 2
_BN = 400
_EPS = 1e-5


def _sc_mesh():
    return plsc.VectorSubcoreMesh(core_axis_name="c", subcore_axis_name="s")


def _pipelined_group(x_hbm, src_v, dst_v, rows_v, acc_sh, gsems, ssems):
    del ssems
    nsub = _GRP
    handles = [None] * nsub

    def fire(t):
        slot = t % _NSLOT
        return pltpu.async_copy(x_hbm.at[src_v.at[t]],
                                rows_v.at[slot], gsems[slot])

    for t in range(min(_NSLOT, nsub)):
        handles[t] = fire(t)
    for t in range(nsub):
        handles[t].wait()
        pltpu.sync_copy(rows_v.at[t % _NSLOT],
                        acc_sh.at[dst_v.at[t]], add=True)
        if t + _NSLOT < nsub:
            handles[t + _NSLOT] = fire(t + _NSLOT)


def _sc_layer0(x, src_a, dst_a, zrows, ones128):
    @functools.partial(
        pl.kernel,
        out_type=(
            jax.ShapeDtypeStruct((2, _N, 128), jnp.float32),
            jax.ShapeDtypeStruct((2, _N, 128), jnp.float32),
        ),
        mesh=_sc_mesh(),
        scratch_types=[
            pltpu.VMEM((_GRP, _BATCH), jnp.int32),
            pltpu.VMEM((_GRP, _BATCH), jnp.int32),
            pltpu.VMEM((_NSLOT * _R, _BATCH, 128), jnp.float32),
            pltpu.VMEM_SHARED((_NPAD, 128), jnp.float32),
            pltpu.SemaphoreType.DMA,
            pltpu.SemaphoreType.DMA,
            pltpu.SemaphoreType.DMA,
            pltpu.SemaphoreType.DMA,
        ],
    )
    def k(x_hbm, src_hbm, dst_hbm, zr_hbm, o_hbm,
          agg_out, cnt_out, src_v, dst_v, rows_v,
          acc_sh, gsA, gsB, ssA, ssB):
        c = lax.axis_index("c")
        s = lax.axis_index("s")
        wid = c * 16 + s
        r0 = s * _RPS

        pltpu.sync_copy(zr_hbm, acc_sh.at[pl.ds(r0, _RPS)])
        plsc.subcore_barrier()

        def group(g, carry):
            pltpu.sync_copy(src_hbm.at[wid, pl.ds(g * _GRP, _GRP)], src_v)
            pltpu.sync_copy(dst_hbm.at[wid, pl.ds(g * _GRP, _GRP)], dst_v)
            _pipelined_group(x_hbm, src_v, dst_v, rows_v, acc_sh,
                             (gsA, gsB), (ssA, ssB))
            return carry

        lax.fori_loop(0, _NB0 // _GRP, group, 0)
        plsc.subcore_barrier()

        @pl.when(s < 15)
        def _():
            pltpu.sync_copy(acc_sh.at[pl.ds(r0, _RPS)],
                            agg_out.at[c, pl.ds(r0, _RPS)])

        @pl.when(s == 15)
        def _():
            pltpu.sync_copy(acc_sh.at[pl.ds(15 * _RPS, _N - 15 * _RPS)],
                            agg_out.at[c, pl.ds(15 * _RPS, _N - 15 * _RPS)])

        pltpu.sync_copy(zr_hbm, acc_sh.at[pl.ds(r0, _RPS)])
        pltpu.sync_copy(o_hbm, rows_v.at[0])
        plsc.subcore_barrier()

        def group_cnt(g, carry):
            pltpu.sync_copy(dst_hbm.at[wid, pl.ds(g * _GRP, _GRP)], dst_v)
            for b in range(_GRP):
                pltpu.sync_copy(rows_v.at[0], acc_sh.at[dst_v.at[b]],
                                add=True)
            return carry

        lax.fori_loop(0, _NB0 // _GRP, group_cnt, 0)
        plsc.subcore_barrier()

        @pl.when(s < 15)
        def _():
            pltpu.sync_copy(acc_sh.at[pl.ds(r0, _RPS)],
                            cnt_out.at[c, pl.ds(r0, _RPS)])

        @pl.when(s == 15)
        def _():
            pltpu.sync_copy(acc_sh.at[pl.ds(15 * _RPS, _N - 15 * _RPS)],
                            cnt_out.at[c, pl.ds(15 * _RPS, _N - 15 * _RPS)])

    return k(x, src_a, dst_a, zrows, ones128)


def _sc_spmm(h2d, src_adj, dst_d, zrows):
    @functools.partial(
        pl.kernel,
        out_type=jax.ShapeDtypeStruct((_CH, _N, 128), jnp.float32),
        mesh=_sc_mesh(),
        scratch_types=[
            pltpu.VMEM((_GRP, _BATCH), jnp.int32),
            pltpu.VMEM((_GRP, _BATCH), jnp.int32),
            pltpu.VMEM((_NSLOT * _R, _BATCH, 128), jnp.float32),
            pltpu.VMEM_SHARED((_NPAD, 128), jnp.float32),
            pltpu.SemaphoreType.DMA,
            pltpu.SemaphoreType.DMA,
            pltpu.SemaphoreType.DMA,
            pltpu.SemaphoreType.DMA,
        ],
    )
    def k(h_hbm, src_hbm, dst_hbm, zr_hbm, agg_out,
          src_v, dst_v, rows_v, acc_sh, gsA, gsB, ssA, ssB):
        c = lax.axis_index("c")
        s = lax.axis_index("s")
        r0 = s * _RPS

        for kk in range(_CH // 2):
            ci = c * (_CH // 2) + kk
            pltpu.sync_copy(zr_hbm, acc_sh.at[pl.ds(r0, _RPS)])
            plsc.subcore_barrier()

            def group(g, carry):
                pltpu.sync_copy(src_hbm.at[ci, s, pl.ds(g * _GRP, _GRP)],
                                src_v)
                pltpu.sync_copy(dst_hbm.at[s, pl.ds(g * _GRP, _GRP)], dst_v)
                _pipelined_group(h_hbm, src_v, dst_v, rows_v, acc_sh,
                                 (gsA, gsB), (ssA, ssB))
                return carry

            lax.fori_loop(0, _NBD // _GRP, group, 0)
            plsc.subcore_barrier()

            @pl.when(s < 15)
            def _():
                pltpu.sync_copy(acc_sh.at[pl.ds(r0, _RPS)],
                                agg_out.at[ci, pl.ds(r0, _RPS)])

            @pl.when(s == 15)
            def _():
                pltpu.sync_copy(acc_sh.at[pl.ds(15 * _RPS, _N - 15 * _RPS)],
                                agg_out.at[ci, pl.ds(15 * _RPS, _N - 15 * _RPS)])

    return k(h2d, src_adj, dst_d, zrows)


def _ln_leaky(h, g, b):
    mu = jnp.mean(h, axis=-1, keepdims=True)
    var = jnp.mean((h - mu) ** 2, axis=-1, keepdims=True)
    h = (h - mu) * lax.rsqrt(var + _EPS) * g + b
    return jnp.where(h >= 0, h, 0.01 * h)


def _tc_layer0(aggp, cntp, w, b, vn, g, be):
    def body(p_ref, c_ref, w_ref, b_ref, vn_ref, g_ref, be_ref,
             h_ref, cs_ref, rd_ref):
        i = pl.program_id(0)
        agg = p_ref[0] + p_ref[1]
        cnt = c_ref[0, :, 0:1] + c_ref[1, :, 0:1]
        rdeg = 1.0 / jnp.maximum(cnt, 1.0)
        h = jnp.dot(agg * rdeg, w_ref[...], preferred_element_type=jnp.float32)
        vnm = jnp.mean(vn_ref[...], axis=0, keepdims=True)
        h = _ln_leaky(h + b_ref[...] + vnm, g_ref[...], be_ref[...])
        for cc in range(_CH):
            h_ref[cc] = h[:, cc * 128:(cc + 1) * 128]
        rd_ref[...] = jnp.broadcast_to(rdeg, (_BN, 128))

        @pl.when(i == 0)
        def _():
            cs_ref[...] = jnp.zeros((1, _H), jnp.float32)

        cs_ref[...] += jnp.sum(h, axis=0, keepdims=True)

    return pl.pallas_call(
        body,
        grid=(_N // _BN,),
        in_specs=[
            pl.BlockSpec((2, _BN, 128), lambda i: (0, i, 0)),
            pl.BlockSpec((2, _BN, 128), lambda i: (0, i, 0)),
            pl.BlockSpec((128, _H), lambda i: (0, 0)),
            pl.BlockSpec((1, _H), lambda i: (0, 0)),
            pl.BlockSpec((3, _H), lambda i: (0, 0)),
            pl.BlockSpec((1, _H), lambda i: (0, 0)),
            pl.BlockSpec((1, _H), lambda i: (0, 0)),
        ],
        out_specs=[
            pl.BlockSpec((_CH, _BN, 128), lambda i: (0, i, 0)),
            pl.BlockSpec((1, _H), lambda i: (0, 0)),
            pl.BlockSpec((_BN, 128), lambda i: (i, 0)),
        ],
        out_shape=[
            jax.ShapeDtypeStruct((_CH, _N, 128), jnp.float32),
            jax.ShapeDtypeStruct((1, _H), jnp.float32),
            jax.ShapeDtypeStruct((_N, 128), jnp.float32),
        ],
    )(aggp, cntp, w, b, vn, g, be)


def _tc_hidden(agg, rdeg, w, b, vn, g, be, final, out_w=None, out_b=None):
    def body(*refs):
        if final:
            (a_ref, rd_ref, w_ref, b_ref, vn_ref, g_ref, be_ref,
             ow_ref, ob_ref, y_ref) = refs
        else:
            (a_ref, rd_ref, w_ref, b_ref, vn_ref, g_ref, be_ref,
             h_ref, cs_ref) = refs
        i = pl.program_id(0)
        rd = rd_ref[:, 0:1]
        h = jnp.zeros((_BN, _H), jnp.float32)
        for cc in range(_CH):
            h += jnp.dot(a_ref[cc] * rd, w_ref[cc * 128:(cc + 1) * 128, :],
                         preferred_element_type=jnp.float32)
        vnm = jnp.mean(vn_ref[...], axis=0, keepdims=True)
        h = _ln_leaky(h + b_ref[...] + vnm, g_ref[...], be_ref[...])
        if final:
            y_ref[...] = (jnp.dot(h, ow_ref[...],
                                  preferred_element_type=jnp.float32)
                          + ob_ref[...])
        else:
            for cc in range(_CH):
                h_ref[cc] = h[:, cc * 128:(cc + 1) * 128]

            @pl.when(i == 0)
            def _():
                cs_ref[...] = jnp.zeros((1, _H), jnp.float32)

            cs_ref[...] += jnp.sum(h, axis=0, keepdims=True)

    in_specs = [
        pl.BlockSpec((_CH, _BN, 128), lambda i: (0, i, 0)),
        pl.BlockSpec((_BN, 128), lambda i: (i, 0)),
        pl.BlockSpec((_H, _H), lambda i: (0, 0)),
        pl.BlockSpec((1, _H), lambda i: (0, 0)),
        pl.BlockSpec((3, _H), lambda i: (0, 0)),
        pl.BlockSpec((1, _H), lambda i: (0, 0)),
        pl.BlockSpec((1, _H), lambda i: (0, 0)),
    ]
    args = [agg, rdeg, w, b, vn, g, be]
    if final:
        in_specs += [pl.BlockSpec((_H, 128), lambda i: (0, 0)),
                     pl.BlockSpec((1, 128), lambda i: (0, 0))]
        args += [out_w, out_b]
        out_specs = pl.BlockSpec((_BN, 128), lambda i: (i, 0))
        out_shape = jax.ShapeDtypeStruct((_N, 128), jnp.float32)
    else:
        out_specs = [
            pl.BlockSpec((_CH, _BN, 128), lambda i: (0, i, 0)),
            pl.BlockSpec((1, _H), lambda i: (0, 0)),
        ]
        out_shape = [
            jax.ShapeDtypeStruct((_CH, _N, 128), jnp.float32),
            jax.ShapeDtypeStruct((1, _H), jnp.float32),
        ]
    return pl.pallas_call(
        body, grid=(_N // _BN,), in_specs=in_specs,
        out_specs=out_specs, out_shape=out_shape,
    )(*args)


def _tc_vn_update(cs, vn, w1c, b1r, g1, be1, w2c, b2r, g2, be2):
    def body(cs_ref, vn_ref, w1_ref, b1_ref, g1_ref, be1_ref,
             w2_ref, b2_ref, g2_ref, be2_ref, o_ref):
        gi = cs_ref[...] * (1.0 / _N)
        gmat = vn_ref[...] + gi
        t_all = jnp.dot(gmat, w1_ref[...], preferred_element_type=jnp.float32)
        rows = lax.broadcasted_iota(jnp.int32, (3, 1), 0)
        t = jnp.zeros((3, 128), jnp.float32)
        for v in range(3):
            m = jnp.where(rows == v, 1.0, 0.0)
            t += t_all[:, v * 128:(v + 1) * 128] * m
        t = t + b1_ref[...]
        mu = jnp.mean(t, axis=-1, keepdims=True)
        var = jnp.mean((t - mu) ** 2, axis=-1, keepdims=True)
        t = (t - mu) * lax.rsqrt(var + _EPS) * g1_ref[...] + be1_ref[...]
        t = jnp.maximum(t, 0.0)
        z = jnp.concatenate(
            [t * jnp.where(rows == v, 1.0, 0.0) for v in range(3)], axis=1)
        u = jnp.dot(z, w2_ref[...], preferred_element_type=jnp.float32)
        u = u + b2_ref[...]
        mu = jnp.mean(u, axis=-1, keepdims=True)
        var = jnp.mean((u - mu) ** 2, axis=-1, keepdims=True)
        u = (u - mu) * lax.rsqrt(var + _EPS) * g2_ref[...] + be2_ref[...]
        u = jnp.maximum(u, 0.0)
        o_ref[...] = vn_ref[...] + u

    return pl.pallas_call(
        body,
        out_shape=jax.ShapeDtypeStruct((3, _H), jnp.float32),
    )(cs, vn, w1c, b1r, g1, be1, w2c, b2r, g2, be2)


def _pad_edges(src, dst, n_workers, nb):
    total = n_workers * nb * _BATCH
    pad = total - _E
    psrc = (jnp.arange(pad, dtype=jnp.int32) * 997) % _N
    pdst = _N + (jnp.arange(pad, dtype=jnp.int32) % 16)
    src_p = jnp.concatenate([src, psrc]).reshape(n_workers, nb, _BATCH)
    dst_p = jnp.concatenate([dst, pdst]).reshape(n_workers, nb, _BATCH)
    return src_p, dst_p


def kernel(x, edge_index, params):
    src = edge_index[0].astype(jnp.int32)
    dst = edge_index[1].astype(jnp.int32)

    src_a, dst_a = _pad_edges(src, dst, 32, _NB0)
    src_d, dst_d = _pad_edges(src, dst, 16, _NBD)
    chunk_off = (jnp.arange(_CH, dtype=jnp.int32) * _N).reshape(_CH, 1, 1, 1)
    src_adj = src_d[None] + chunk_off

    zrows = jnp.zeros((_RPS, 128), jnp.float32)
    ones128 = jnp.ones((_BATCH, 128), jnp.float32)

    conv = params["conv"]
    ln = params["ln"]
    vn = params["vn_emb"]
    row = lambda a: a.reshape(1, -1)

    mlp_stacks = []
    for i in range(3):
        ms = params["mlps"][i]
        w1c = jnp.concatenate([m["W1"] for m in ms], axis=1)
        b1r = jnp.stack([m["b1"] for m in ms], axis=0)
        g1 = jnp.stack([m["g1"] for m in ms], axis=0)
        be1 = jnp.stack([m["be1"] for m in ms], axis=0)
        w2c = jnp.concatenate([m["W2"] for m in ms], axis=0)
        b2r = jnp.stack([m["b2"] for m in ms], axis=0)
        g2 = jnp.stack([m["g2"] for m in ms], axis=0)
        be2 = jnp.stack([m["be2"] for m in ms], axis=0)
        mlp_stacks.append((w1c, b1r, g1, be1, w2c, b2r, g2, be2))

    aggp, cntp = _sc_layer0(x, src_a, dst_a, zrows, ones128)
    hc, cs, rdeg = _tc_layer0(aggp, cntp, conv[0]["W"], row(conv[0]["b"]),
                              vn, row(ln[0]["g"]), row(ln[0]["b"]))
    vn = _tc_vn_update(cs, vn, *mlp_stacks[0])

    for i in (1, 2):
        agg = _sc_spmm(hc.reshape(_CH * _N, 128), src_adj, dst_d, zrows)
        hc, cs = _tc_hidden(agg, rdeg, conv[i]["W"], row(conv[i]["b"]),
                            vn, row(ln[i]["g"]), row(ln[i]["b"]), final=False)
        vn = _tc_vn_update(cs, vn, *mlp_stacks[i])

    agg = _sc_spmm(hc.reshape(_CH * _N, 128), src_adj, dst_d, zrows)
    return _tc_hidden(agg, rdeg, conv[3]["W"], row(conv[3]["b"]),
                      vn, row(ln[3]["g"]), row(ln[3]["b"]), final=True,
                      out_w=params["out_W"], out_b=row(params["out_b"]))
